# baseline (speedup 1.0000x reference)
import jax
import jax.numpy as jnp
from jax import lax
from jax.experimental import pallas as pl
from jax.experimental.pallas import tpu as pltpu
from jax.experimental.pallas import tpu_sc as plsc

_NC = 2
_NS = 16
_NW = _NC * _NS

_CHUNK = 512
_IPG = 128
_GPC = _CHUNK // _IPG


def _body(idx_hbm, table_hbm, out_hbm, idx_v, rows_v, sem):
    num_chunks = idx_hbm.shape[0]
    chunks_per_w = num_chunks // _NW
    wid = lax.axis_index("s") * _NC + lax.axis_index("c")

    def step(g, carry):
        chunk = wid * chunks_per_w + g
        pltpu.sync_copy(idx_hbm.at[chunk], idx_v)
        copies = [
            pltpu.async_copy(
                table_hbm.at[idx_v.at[j]],
                rows_v.at[pl.ds(j * _IPG, _IPG)],
                sem,
            )
            for j in range(_GPC)
        ]
        for cp in copies:
            cp.wait()
        base = pl.multiple_of(chunk * _CHUNK, _CHUNK)
        pltpu.sync_copy(rows_v, out_hbm.at[pl.ds(base, _CHUNK)])
        return carry

    lax.fori_loop(0, chunks_per_w, step, 0)


def kernel(token_ids, W):
    S, T = token_ids.shape
    D = W.shape[1]
    B = S * T
    num_chunks = B // _CHUNK
    idx = token_ids.astype(jnp.int32).reshape(num_chunks, _GPC, _IPG)

    mesh = plsc.VectorSubcoreMesh(core_axis_name="c", subcore_axis_name="s")
    run = pl.kernel(
        _body,
        out_type=jax.ShapeDtypeStruct((B, D), jnp.float32),
        mesh=mesh,
        scratch_types=[
            pltpu.VMEM((_GPC, _IPG), jnp.int32),
            pltpu.VMEM((_CHUNK, D), jnp.float32),
            pltpu.SemaphoreType.DMA,
        ],
        compiler_params=pltpu.CompilerParams(use_tc_tiling_on_sc=False),
    )
    out = run(idx, W)
    return out.reshape(S, T, D)


# speedup vs baseline: 1.0300x; 1.0300x over previous
import jax
import jax.numpy as jnp
from jax import lax
from jax.experimental import pallas as pl
from jax.experimental.pallas import tpu as pltpu
from jax.experimental.pallas import tpu_sc as plsc

_NC = 2
_NS = 16
_NW = _NC * _NS

_CHUNK = 256
_IPG = 128
_GPC = _CHUNK // _IPG
_RING = 4


def _body(idx_hbm, table_hbm, out_hbm, idx_v, rows_v, gsem, osem):
    cpw = idx_hbm.shape[1] // _GPC
    wid = lax.axis_index("s") * _NC + lax.axis_index("c")
    pltpu.sync_copy(idx_hbm.at[wid], idx_v)
    out_base = wid * cpw

    def fire_gathers(g, b):
        for j in range(_GPC):
            pltpu.async_copy(
                table_hbm.at[idx_v.at[g * _GPC + j]],
                rows_v.at[b, pl.ds(j * _IPG, _IPG)],
                gsem.at[b],
            )

    def drain_gathers(b):
        pltpu.make_async_copy(
            out_hbm.at[pl.ds(0, _CHUNK)], rows_v.at[b], gsem.at[b]
        ).wait()

    def out_slice(g):
        base = pl.multiple_of((out_base + g) * _CHUNK, _CHUNK)
        return out_hbm.at[pl.ds(base, _CHUNK)]

    for b in range(_RING):
        fire_gathers(b, b)

    @pl.loop(0, cpw - _RING, step=_RING)
    def _steady(i):
        for b in range(_RING):
            g = i + b
            drain_gathers(b)
            pltpu.async_copy(rows_v.at[b], out_slice(g), osem.at[b])
            pltpu.make_async_copy(
                rows_v.at[b], out_hbm.at[pl.ds(0, _CHUNK)], osem.at[b]
            ).wait()
            fire_gathers(g + _RING, b)

    for b in range(_RING):
        g = cpw - _RING + b
        drain_gathers(b)
        pltpu.sync_copy(rows_v.at[b], out_slice(g))


def kernel(token_ids, W):
    S, T = token_ids.shape
    D = W.shape[1]
    B = S * T
    rows_per_w = B // _NW
    idx = token_ids.astype(jnp.int32).reshape(_NW, rows_per_w // _IPG, _IPG)

    mesh = plsc.VectorSubcoreMesh(core_axis_name="c", subcore_axis_name="s")
    run = pl.kernel(
        _body,
        out_type=jax.ShapeDtypeStruct((B, D), jnp.float32),
        mesh=mesh,
        scratch_types=[
            pltpu.VMEM((rows_per_w // _IPG, _IPG), jnp.int32),
            pltpu.VMEM((_RING, _CHUNK, D), jnp.float32),
            pltpu.SemaphoreType.DMA((_RING,)),
            pltpu.SemaphoreType.DMA((_RING,)),
        ],
        compiler_params=pltpu.CompilerParams(use_tc_tiling_on_sc=False),
    )
    out = run(idx, W)
    return out.reshape(S, T, D)
